# baseline (speedup 1.0000x reference)
import functools

import jax
import jax.numpy as jnp
from jax.experimental import pallas as pl
from jax.experimental.pallas import tpu as pltpu

GRID = 24
RES = 0.25
HALF = GRID * RES / 2.0
NCELL = GRID * GRID
NPAD = 640
CHUNK = 4096
F = 128
C1 = 64
C2 = 32
POOL = 4
PG = GRID // POOL
EMB = C2 * PG * PG
NW = 28


def _fused_kernel(x_ref, y_ref, v_ref, encw_ref, encb_ref, *refs,
                  num_chunks):
    hbm = refs[:NW]
    clf_ref, reg_ref, cyc_ref = refs[NW:NW + 3]
    acc_ref = refs[NW + 3]
    wvm = refs[NW + 4:NW + 4 + NW]
    sem = refs[NW + 4 + NW]

    i = pl.program_id(0)

    def copies():
        return [pltpu.make_async_copy(hbm[j], wvm[j], sem.at[j])
                for j in range(NW)]

    @pl.when(i == 0)
    def _init():
        acc_ref[...] = jnp.zeros_like(acc_ref)
        for c in copies():
            c.start()

    x = x_ref[0]
    y = y_ref[0]
    v = v_ref[0]
    cx = ((x + HALF) / RES).astype(jnp.int32)
    cy = ((y + HALF) / RES).astype(jnp.int32)
    mask = (cx >= 0) & (cx < GRID) & (cy >= 0) & (cy < GRID)
    idx = jnp.where(mask, cx * GRID + cy, NCELL)

    encw = jnp.transpose(encw_ref[...])
    encb = jnp.transpose(encb_ref[...])
    featT = jax.nn.relu(encw * v + encb)

    cell_ids = jax.lax.broadcasted_iota(jnp.int32, (NPAD, CHUNK), 0)
    onehotT = (cell_ids == idx).astype(jnp.float32)

    acc_ref[...] += jax.lax.dot_general(
        featT, onehotT, (((1,), (1,)), ((), ())),
        precision=jax.lax.Precision.HIGHEST,
        preferred_element_type=jnp.float32)

    @pl.when(i == num_chunks - 1)
    def _tail():
        for c in copies():
            c.wait()
        w1flat, b1, w2flat, b2 = (wvm[0], wvm[1], wvm[2], wvm[3])

        gt = acc_ref[:, :NCELL]

        r = jax.lax.broadcasted_iota(jnp.int32, (1, NCELL), 1)
        p = r // GRID
        q = r - p * GRID

        def conv(src, wflat_ref, b_ref, cin, cout):
            kk = jax.lax.broadcasted_iota(jnp.int32, (cin * 9, cin), 0)
            cc = jax.lax.broadcasted_iota(jnp.int32, (cin * 9, cin), 1)
            h = jnp.zeros((cout, NCELL), dtype=jnp.float32)
            for dd in range(9):
                di, dj = dd // 3 - 1, dd % 3 - 1
                off = di * GRID + dj
                valid = ((p + di >= 0) & (p + di < GRID) &
                         (q + dj >= 0) & (q + dj < GRID))
                shifted = jnp.roll(src, -off, axis=1) if off else src
                shifted = jnp.where(valid, shifted, 0.0)
                sel = (kk == cc * 9 + dd).astype(jnp.float32)
                wdd = jnp.dot(wflat_ref[...], sel,
                              preferred_element_type=jnp.float32)
                h = h + jnp.dot(wdd, shifted,
                                preferred_element_type=jnp.float32)
            bias = jnp.transpose(b_ref[...])
            return jax.nn.relu(h + bias)

        h1 = conv(gt, w1flat, b1, F, C1)
        h2 = conv(h1, w2flat, b2, C1, C2)

        rr = jax.lax.broadcasted_iota(jnp.int32, (NCELL, PG * PG), 0)
        ss = jax.lax.broadcasted_iota(jnp.int32, (NCELL, PG * PG), 1)
        pm = ((rr // (GRID * POOL) == ss // PG) &
              ((rr % GRID) // POOL == ss % PG))
        pmat = pm.astype(jnp.float32) * (1.0 / (POOL * POOL))
        pooled = jnp.dot(h2, pmat, preferred_element_type=jnp.float32)

        emb = jnp.concatenate([pooled[c:c + 1, :] for c in range(C2)], axis=1)

        def head(h_idx, nout):
            o = 4 + h_idx * 6
            h = jax.nn.relu(jnp.dot(emb, wvm[o][...],
                                    preferred_element_type=jnp.float32)
                            + wvm[o + 1][...])
            h = jax.nn.relu(jnp.dot(h, wvm[o + 2][...],
                                    preferred_element_type=jnp.float32)
                            + wvm[o + 3][...])
            return (jnp.dot(h, wvm[o + 4][...],
                            preferred_element_type=jnp.float32)
                    + wvm[o + 5][...])

        clf_ref[...] = head(0, 8)
        reg_ref[...] = head(1, 6)
        sin_o = jnp.tanh(head(2, 1))
        cos_o = jnp.tanh(head(3, 1))
        cyc_ref[...] = jnp.concatenate([sin_o, cos_o], axis=1)


def kernel(ventana, params):
    nwin, npts, _ = ventana.shape
    num_chunks = nwin * npts // CHUNK
    x = ventana[:, :, 0].reshape(num_chunks, 1, CHUNK)
    y = ventana[:, :, 1].reshape(num_chunks, 1, CHUNK)
    v = ventana[:, :, 3].reshape(num_chunks, 1, CHUNK)

    encw = params["enc"][0]
    encb = params["enc"][1].reshape(1, F)

    weights = [params["conv1"][0].reshape(C1, F * 9),
               params["conv1"][1].reshape(1, C1),
               params["conv2"][0].reshape(C2, C1 * 9),
               params["conv2"][1].reshape(1, C2)]
    for name in ("clf", "reg", "sin", "cos"):
        for w, b in params[name]:
            weights.append(w)
            weights.append(b.reshape(1, -1))
    assert len(weights) == NW

    chunk_spec = pl.BlockSpec((1, 1, CHUNK), lambda i: (i, 0, 0))
    full = lambda a: pl.BlockSpec(a.shape, lambda i: (0,) * a.ndim)
    any_spec = pl.BlockSpec(memory_space=pltpu.MemorySpace.HBM)

    logits, reg_out, cyc_out = pl.pallas_call(
        functools.partial(_fused_kernel, num_chunks=num_chunks),
        grid=(num_chunks,),
        in_specs=[chunk_spec, chunk_spec, chunk_spec,
                  full(encw), full(encb)] + [any_spec] * NW,
        out_specs=(pl.BlockSpec((1, 8), lambda i: (0, 0)),
                   pl.BlockSpec((1, 6), lambda i: (0, 0)),
                   pl.BlockSpec((1, 2), lambda i: (0, 0))),
        out_shape=(jax.ShapeDtypeStruct((1, 8), jnp.float32),
                   jax.ShapeDtypeStruct((1, 6), jnp.float32),
                   jax.ShapeDtypeStruct((1, 2), jnp.float32)),
        scratch_shapes=([pltpu.VMEM((F, NPAD), jnp.float32)] +
                        [pltpu.VMEM(w.shape, jnp.float32) for w in weights] +
                        [pltpu.SemaphoreType.DMA((NW,))]),
    )(x, y, v, encw, encb, *weights)

    return (logits, reg_out, cyc_out)


# speedup vs baseline: 2.4339x; 2.4339x over previous
import functools

import jax
import jax.numpy as jnp
from jax.experimental import pallas as pl
from jax.experimental.pallas import tpu as pltpu

GRID = 24
RES = 0.25
HALF = GRID * RES / 2.0
NCELL = GRID * GRID
NPAD = 640
CHUNK = 4096
F = 128
C1 = 64
C2 = 32
POOL = 4
PG = GRID // POOL
EMB = C2 * PG * PG
NW = 28


def _fused_kernel(x_ref, y_ref, v_ref, encw_ref, encb_ref, *refs,
                  num_chunks):
    hbm = refs[:NW]
    clf_ref, reg_ref, cyc_ref = refs[NW:NW + 3]
    acc_ref = refs[NW + 3]
    wvm = refs[NW + 4:NW + 4 + NW]
    sem = refs[NW + 4 + NW]

    i = pl.program_id(0)

    def copies():
        return [pltpu.make_async_copy(hbm[j], wvm[j], sem.at[j])
                for j in range(NW)]

    @pl.when(i == 0)
    def _init():
        acc_ref[...] = jnp.zeros_like(acc_ref)
        for c in copies():
            c.start()

    x = x_ref[0]
    y = y_ref[0]
    v = v_ref[0]
    cx = ((x + HALF) / RES).astype(jnp.int32)
    cy = ((y + HALF) / RES).astype(jnp.int32)
    mask = (cx >= 0) & (cx < GRID) & (cy >= 0) & (cy < GRID)
    idx = jnp.where(mask, cx * GRID + cy, NCELL)

    encw = jnp.transpose(encw_ref[...])
    encb = jnp.transpose(encb_ref[...])
    featT = jax.nn.relu(encw * v + encb)

    cell_ids = jax.lax.broadcasted_iota(jnp.int32, (NPAD, CHUNK), 0)
    onehotT = (cell_ids == idx).astype(jnp.float32)

    acc_ref[...] += jax.lax.dot_general(
        featT, onehotT, (((1,), (1,)), ((), ())),
        preferred_element_type=jnp.float32)

    @pl.when(i == num_chunks - 1)
    def _tail():
        for c in copies():
            c.wait()
        w1flat, b1, w2flat, b2 = (wvm[0], wvm[1], wvm[2], wvm[3])

        gt = acc_ref[:, :NCELL]

        r = jax.lax.broadcasted_iota(jnp.int32, (1, NCELL), 1)
        p = r // GRID
        q = r - p * GRID

        def conv(src, wflat_ref, b_ref, cin, cout):
            kk = jax.lax.broadcasted_iota(jnp.int32, (cin * 9, cin), 0)
            cc = jax.lax.broadcasted_iota(jnp.int32, (cin * 9, cin), 1)
            h = jnp.zeros((cout, NCELL), dtype=jnp.float32)
            for dd in range(9):
                di, dj = dd // 3 - 1, dd % 3 - 1
                off = di * GRID + dj
                valid = ((p + di >= 0) & (p + di < GRID) &
                         (q + dj >= 0) & (q + dj < GRID))
                shifted = jnp.roll(src, -off, axis=1) if off else src
                shifted = jnp.where(valid, shifted, 0.0)
                sel = (kk == cc * 9 + dd).astype(jnp.float32)
                wdd = jnp.dot(wflat_ref[...], sel,
                              precision=jax.lax.Precision.HIGHEST,
                              preferred_element_type=jnp.float32)
                h = h + jnp.dot(wdd, shifted,
                                precision=jax.lax.Precision.HIGHEST,
                                preferred_element_type=jnp.float32)
            bias = jnp.transpose(b_ref[...])
            return jax.nn.relu(h + bias)

        h1 = conv(gt, w1flat, b1, F, C1)
        h2 = conv(h1, w2flat, b2, C1, C2)

        rr = jax.lax.broadcasted_iota(jnp.int32, (NCELL, PG * PG), 0)
        ss = jax.lax.broadcasted_iota(jnp.int32, (NCELL, PG * PG), 1)
        pm = ((rr // (GRID * POOL) == ss // PG) &
              ((rr % GRID) // POOL == ss % PG))
        pmat = pm.astype(jnp.float32) * (1.0 / (POOL * POOL))
        pooled = jnp.dot(h2, pmat, precision=jax.lax.Precision.HIGHEST,
                         preferred_element_type=jnp.float32)

        emb = jnp.concatenate([pooled[c:c + 1, :] for c in range(C2)], axis=1)

        def head(h_idx, nout):
            o = 4 + h_idx * 6
            hp = jax.lax.Precision.HIGHEST
            h = jax.nn.relu(jnp.dot(emb, wvm[o][...], precision=hp,
                                    preferred_element_type=jnp.float32)
                            + wvm[o + 1][...])
            h = jax.nn.relu(jnp.dot(h, wvm[o + 2][...], precision=hp,
                                    preferred_element_type=jnp.float32)
                            + wvm[o + 3][...])
            return (jnp.dot(h, wvm[o + 4][...], precision=hp,
                            preferred_element_type=jnp.float32)
                    + wvm[o + 5][...])

        clf_ref[...] = head(0, 8)
        reg_ref[...] = head(1, 6)
        sin_o = jnp.tanh(head(2, 1))
        cos_o = jnp.tanh(head(3, 1))
        cyc_ref[...] = jnp.concatenate([sin_o, cos_o], axis=1)


def kernel(ventana, params):
    nwin, npts, _ = ventana.shape
    num_chunks = nwin * npts // CHUNK
    x = ventana[:, :, 0].reshape(num_chunks, 1, CHUNK)
    y = ventana[:, :, 1].reshape(num_chunks, 1, CHUNK)
    v = ventana[:, :, 3].reshape(num_chunks, 1, CHUNK)

    encw = params["enc"][0]
    encb = params["enc"][1].reshape(1, F)

    weights = [params["conv1"][0].reshape(C1, F * 9),
               params["conv1"][1].reshape(1, C1),
               params["conv2"][0].reshape(C2, C1 * 9),
               params["conv2"][1].reshape(1, C2)]
    for name in ("clf", "reg", "sin", "cos"):
        for w, b in params[name]:
            weights.append(w)
            weights.append(b.reshape(1, -1))
    assert len(weights) == NW

    chunk_spec = pl.BlockSpec((1, 1, CHUNK), lambda i: (i, 0, 0))
    full = lambda a: pl.BlockSpec(a.shape, lambda i: (0,) * a.ndim)
    any_spec = pl.BlockSpec(memory_space=pltpu.MemorySpace.HBM)

    logits, reg_out, cyc_out = pl.pallas_call(
        functools.partial(_fused_kernel, num_chunks=num_chunks),
        grid=(num_chunks,),
        in_specs=[chunk_spec, chunk_spec, chunk_spec,
                  full(encw), full(encb)] + [any_spec] * NW,
        out_specs=(pl.BlockSpec((1, 8), lambda i: (0, 0)),
                   pl.BlockSpec((1, 6), lambda i: (0, 0)),
                   pl.BlockSpec((1, 2), lambda i: (0, 0))),
        out_shape=(jax.ShapeDtypeStruct((1, 8), jnp.float32),
                   jax.ShapeDtypeStruct((1, 6), jnp.float32),
                   jax.ShapeDtypeStruct((1, 2), jnp.float32)),
        scratch_shapes=([pltpu.VMEM((F, NPAD), jnp.float32)] +
                        [pltpu.VMEM(w.shape, jnp.float32) for w in weights] +
                        [pltpu.SemaphoreType.DMA((NW,))]),
    )(x, y, v, encw, encb, *weights)

    return (logits, reg_out, cyc_out)
